# baseline (speedup 1.0000x reference)
import jax
import jax.numpy as jnp
from jax import lax
from jax.experimental import pallas as pl
from jax.experimental.pallas import tpu as pltpu
from jax.experimental.pallas import tpu_sc as plsc

_L = 16


def _tracker_update_body(trk_hbm, head_hbm, cmp_hbm, out_hbm,
                         head_v, cmp0_v, cmp1_v, row0_v, row1_v,
                         sem_copy, sem_head, sem_in, sem_out):
    num_sel = head_hbm.shape[0]
    width = cmp_hbm.shape[1]
    wid = lax.axis_index("s") * 2 + lax.axis_index("c")

    bulk = pltpu.async_copy(trk_hbm.at[pl.ds(4 * wid, 4)],
                            out_hbm.at[pl.ds(4 * wid, 4)], sem_copy)
    pltpu.async_copy(head_hbm, head_v, sem_head).wait()

    sel = [jnp.bool_(False), jnp.bool_(False)]
    j = [jnp.int32(0), jnp.int32(0)]
    for c in range(num_sel // _L):
        hv = head_v[pl.ds(c * _L, _L)]
        for i in range(_L):
            h = hv[i]
            for rr in range(2):
                hit = h == 2 * wid + rr
                sel[rr] = sel[rr] | hit
                j[rr] = jnp.where(hit, jnp.int32(c * _L + i), j[rr])

    cmp_bufs = (cmp0_v, cmp1_v)
    row_bufs = (row0_v, row1_v)
    for rr in range(2):
        @pl.when(sel[rr])
        def _(rr=rr):
            pltpu.async_copy(cmp_hbm.at[j[rr]], cmp_bufs[rr], sem_in)
            pltpu.async_copy(trk_hbm.at[2 * (2 * wid + rr)], row_bufs[rr],
                             sem_in)

    neg1 = jnp.full((_L,), -1, jnp.int32)
    bulk.wait()
    for rr in range(2):
        @pl.when(sel[rr])
        def _(rr=rr):
            pltpu.make_async_copy(cmp_hbm.at[j[rr]], cmp_bufs[rr],
                                  sem_in).wait()
            pltpu.make_async_copy(trk_hbm.at[2 * (2 * wid + rr)],
                                  row_bufs[rr], sem_in).wait()

            def mask_body(k, carry):
                base = k * _L
                cv = cmp_bufs[rr][pl.ds(base, _L)]
                tv = row_bufs[rr][pl.ds(base, _L)]
                row_bufs[rr][pl.ds(base, _L)] = jnp.where(cv != -1, neg1, tv)
                return carry

            lax.fori_loop(0, width // _L, mask_body, 0, unroll=8)
            pltpu.async_copy(row_bufs[rr], out_hbm.at[2 * (2 * wid + rr)],
                             sem_out)

    for rr in range(2):
        @pl.when(sel[rr])
        def _(rr=rr):
            pltpu.make_async_copy(row_bufs[rr],
                                  out_hbm.at[2 * (2 * wid + rr)],
                                  sem_out).wait()


def kernel(tracker, head_idx, seq_idx, compute_idx):
    num_heads, row_len = tracker.shape
    num_sel, width = compute_idx.shape
    del seq_idx
    trk2 = tracker.reshape(2 * num_heads, width)

    kern = pl.kernel(
        _tracker_update_body,
        out_type=jax.ShapeDtypeStruct((2 * num_heads, width), jnp.int32),
        mesh=plsc.VectorSubcoreMesh(core_axis_name="c", subcore_axis_name="s"),
        scratch_types=[
            pltpu.VMEM((num_sel,), jnp.int32),
            pltpu.VMEM((width,), jnp.int32),
            pltpu.VMEM((width,), jnp.int32),
            pltpu.VMEM((width,), jnp.int32),
            pltpu.VMEM((width,), jnp.int32),
            pltpu.SemaphoreType.DMA,
            pltpu.SemaphoreType.DMA,
            pltpu.SemaphoreType.DMA,
            pltpu.SemaphoreType.DMA,
        ],
    )
    out2 = kern(trk2, head_idx, compute_idx)
    return out2.reshape(num_heads, row_len)


# speedup vs baseline: 5.4791x; 5.4791x over previous
import jax
import jax.numpy as jnp
from jax import lax
from jax.experimental import pallas as pl
from jax.experimental.pallas import tpu as pltpu
from jax.experimental.pallas import tpu_sc as plsc

_L = 16


def _tracker_update_body(trk_hbm, head_hbm, cmp_hbm, out_hbm,
                         head_v, b0, b1, b2, b3, c0, c1,
                         sem_head, sl0, sl1, sl2, sl3,
                         sc0, sc1, ss0, ss1, ss2, ss3):
    num_sel = head_hbm.shape[0]
    width = cmp_hbm.shape[1]
    wid = lax.axis_index("s") * 2 + lax.axis_index("c")
    base = 4 * wid

    bufs = (b0, b1, b2, b3)
    sem_ld = (sl0, sl1, sl2, sl3)
    sem_st = (ss0, ss1, ss2, ss3)
    cmp_bufs = (c0, c1)
    sem_cmp = (sc0, sc1)

    loads = [pltpu.async_copy(trk_hbm.at[base + h], bufs[h], sem_ld[h])
             for h in range(4)]
    pltpu.async_copy(head_hbm, head_v, sem_head).wait()

    sel = [jnp.bool_(False), jnp.bool_(False)]
    j = [jnp.int32(0), jnp.int32(0)]
    for c in range(num_sel // _L):
        hv = head_v[pl.ds(c * _L, _L)]
        for i in range(_L):
            h = hv[i]
            for rr in range(2):
                hit = h == 2 * wid + rr
                sel[rr] = sel[rr] | hit
                j[rr] = jnp.where(hit, jnp.int32(c * _L + i), j[rr])

    for rr in range(2):
        @pl.when(sel[rr])
        def _(rr=rr):
            pltpu.async_copy(cmp_hbm.at[j[rr]], cmp_bufs[rr], sem_cmp[rr])

    neg1 = jnp.full((_L,), -1, jnp.int32)
    for h in range(4):
        loads[h].wait()
        if h % 2 == 0:
            rr = h // 2

            @pl.when(sel[rr])
            def _(rr=rr, h=h):
                pltpu.make_async_copy(cmp_hbm.at[j[rr]], cmp_bufs[rr],
                                      sem_cmp[rr]).wait()

                def mask_body(k, carry):
                    bs = k * _L
                    cv = cmp_bufs[rr][pl.ds(bs, _L)]
                    tv = bufs[h][pl.ds(bs, _L)]
                    bufs[h][pl.ds(bs, _L)] = jnp.where(cv != -1, neg1, tv)
                    return carry

                lax.fori_loop(0, width // _L, mask_body, 0, unroll=8)

        pltpu.async_copy(bufs[h], out_hbm.at[base + h], sem_st[h])

    for h in range(4):
        pltpu.make_async_copy(bufs[h], out_hbm.at[base + h], sem_st[h]).wait()


def kernel(tracker, head_idx, seq_idx, compute_idx):
    num_heads, row_len = tracker.shape
    num_sel, width = compute_idx.shape
    del seq_idx
    trk2 = tracker.reshape(2 * num_heads, width)

    kern = pl.kernel(
        _tracker_update_body,
        out_type=jax.ShapeDtypeStruct((2 * num_heads, width), jnp.int32),
        mesh=plsc.VectorSubcoreMesh(core_axis_name="c", subcore_axis_name="s"),
        scratch_types=[
            pltpu.VMEM((num_sel,), jnp.int32),
            pltpu.VMEM((width,), jnp.int32),
            pltpu.VMEM((width,), jnp.int32),
            pltpu.VMEM((width,), jnp.int32),
            pltpu.VMEM((width,), jnp.int32),
            pltpu.VMEM((width,), jnp.int32),
            pltpu.VMEM((width,), jnp.int32),
        ] + [pltpu.SemaphoreType.DMA] * 11,
    )
    out2 = kern(trk2, head_idx, compute_idx)
    return out2.reshape(num_heads, row_len)


# speedup vs baseline: 7.8872x; 1.4395x over previous
import jax
import jax.numpy as jnp
from jax import lax
from jax.experimental import pallas as pl
from jax.experimental.pallas import tpu as pltpu
from jax.experimental.pallas import tpu_sc as plsc

_L = 16


def _tracker_update_body(trk_hbm, head_hbm, cmp_hbm, out_hbm,
                         head_v, b0, b1, b2, b3, c0, c1,
                         sem_head, sl0, sl1, sl2, sl3,
                         sc0, sc1, ss0, ss1, ss2, ss3):
    num_sel = head_hbm.shape[0]
    width = cmp_hbm.shape[1]
    wid = lax.axis_index("s") * 2 + lax.axis_index("c")

    bufs = (b0, b1, b2, b3)
    sem_ld = (sl0, sl1, sl2, sl3)
    sem_st = (ss0, ss1, ss2, ss3)
    cmp_bufs = (c0, c1)
    sem_cmp = (sc0, sc1)

    def half_slice(h):
        return (2 * wid + h // 2, pl.ds((h % 2) * width, width))

    loads = [pltpu.async_copy(trk_hbm.at[half_slice(h)], bufs[h], sem_ld[h])
             for h in range(4)]
    pltpu.async_copy(head_hbm, head_v, sem_head).wait()

    sel = [jnp.bool_(False), jnp.bool_(False)]
    j = [jnp.int32(0), jnp.int32(0)]
    for c in range(num_sel // _L):
        hv = head_v[pl.ds(c * _L, _L)]
        for i in range(_L):
            h = hv[i]
            for rr in range(2):
                hit = h == 2 * wid + rr
                sel[rr] = sel[rr] | hit
                j[rr] = jnp.where(hit, jnp.int32(c * _L + i), j[rr])

    for rr in range(2):
        @pl.when(sel[rr])
        def _(rr=rr):
            pltpu.async_copy(cmp_hbm.at[j[rr]], cmp_bufs[rr], sem_cmp[rr])

    neg1 = jnp.full((_L,), -1, jnp.int32)
    for h in range(4):
        loads[h].wait()
        if h % 2 == 0:
            rr = h // 2

            @pl.when(sel[rr])
            def _(rr=rr, h=h):
                pltpu.make_async_copy(cmp_hbm.at[j[rr]], cmp_bufs[rr],
                                      sem_cmp[rr]).wait()

                def mask_body(k, carry):
                    bs = k * _L
                    cv = cmp_bufs[rr][pl.ds(bs, _L)]
                    tv = bufs[h][pl.ds(bs, _L)]
                    bufs[h][pl.ds(bs, _L)] = jnp.where(cv != -1, neg1, tv)
                    return carry

                lax.fori_loop(0, width // _L, mask_body, 0, unroll=8)

        pltpu.async_copy(bufs[h], out_hbm.at[half_slice(h)], sem_st[h])

    for h in range(4):
        pltpu.make_async_copy(bufs[h], out_hbm.at[half_slice(h)],
                              sem_st[h]).wait()


def kernel(tracker, head_idx, seq_idx, compute_idx):
    num_heads, row_len = tracker.shape
    num_sel, width = compute_idx.shape
    del seq_idx

    kern = pl.kernel(
        _tracker_update_body,
        out_type=jax.ShapeDtypeStruct((num_heads, row_len), jnp.int32),
        mesh=plsc.VectorSubcoreMesh(core_axis_name="c", subcore_axis_name="s"),
        scratch_types=[
            pltpu.VMEM((num_sel,), jnp.int32),
            pltpu.VMEM((width,), jnp.int32),
            pltpu.VMEM((width,), jnp.int32),
            pltpu.VMEM((width,), jnp.int32),
            pltpu.VMEM((width,), jnp.int32),
            pltpu.VMEM((width,), jnp.int32),
            pltpu.VMEM((width,), jnp.int32),
        ] + [pltpu.SemaphoreType.DMA] * 11,
    )
    return kern(tracker, head_idx, compute_idx)


# speedup vs baseline: 8.4435x; 1.0705x over previous
import jax
import jax.numpy as jnp
from jax import lax
from jax.experimental import pallas as pl
from jax.experimental.pallas import tpu as pltpu
from jax.experimental.pallas import tpu_sc as plsc

_L = 16


def _tracker_update_body(trk_hbm, head_hbm, cmp_hbm, out_hbm,
                         head_v, b0, b1, c0, c1,
                         sem_head, sl0, sl1, sc0, sc1, ss0, ss1):
    num_sel = head_hbm.shape[0]
    width = cmp_hbm.shape[1]
    wid = lax.axis_index("s") * 2 + lax.axis_index("c")

    bufs = (b0, b1)
    sem_ld = (sl0, sl1)
    sem_st = (ss0, ss1)
    cmp_bufs = (c0, c1)
    sem_cmp = (sc0, sc1)

    loads = [pltpu.async_copy(trk_hbm.at[2 * wid + rr], bufs[rr], sem_ld[rr])
             for rr in range(2)]
    pltpu.async_copy(head_hbm, head_v, sem_head).wait()

    sel = [jnp.bool_(False), jnp.bool_(False)]
    j = [jnp.int32(0), jnp.int32(0)]
    for c in range(num_sel // _L):
        hv = head_v[pl.ds(c * _L, _L)]
        for i in range(_L):
            h = hv[i]
            for rr in range(2):
                hit = h == 2 * wid + rr
                sel[rr] = sel[rr] | hit
                j[rr] = jnp.where(hit, jnp.int32(c * _L + i), j[rr])

    for rr in range(2):
        @pl.when(sel[rr])
        def _(rr=rr):
            pltpu.async_copy(cmp_hbm.at[j[rr]], cmp_bufs[rr], sem_cmp[rr])

    neg1 = jnp.full((_L,), -1, jnp.int32)
    for rr in range(2):
        loads[rr].wait()

        @pl.when(sel[rr])
        def _(rr=rr):
            pltpu.make_async_copy(cmp_hbm.at[j[rr]], cmp_bufs[rr],
                                  sem_cmp[rr]).wait()

            def mask_body(k, carry):
                bs = k * _L
                cv = cmp_bufs[rr][pl.ds(bs, _L)]
                tv = bufs[rr][pl.ds(bs, _L)]
                bufs[rr][pl.ds(bs, _L)] = jnp.where(cv != -1, neg1, tv)
                return carry

            lax.fori_loop(0, width // _L, mask_body, 0, unroll=8)

        pltpu.async_copy(bufs[rr], out_hbm.at[2 * wid + rr], sem_st[rr])

    for rr in range(2):
        pltpu.make_async_copy(bufs[rr], out_hbm.at[2 * wid + rr],
                              sem_st[rr]).wait()


def kernel(tracker, head_idx, seq_idx, compute_idx):
    num_heads, row_len = tracker.shape
    num_sel, width = compute_idx.shape
    del seq_idx

    kern = pl.kernel(
        _tracker_update_body,
        out_type=jax.ShapeDtypeStruct((num_heads, row_len), jnp.int32),
        mesh=plsc.VectorSubcoreMesh(core_axis_name="c", subcore_axis_name="s"),
        scratch_types=[
            pltpu.VMEM((num_sel,), jnp.int32),
            pltpu.VMEM((row_len,), jnp.int32),
            pltpu.VMEM((row_len,), jnp.int32),
            pltpu.VMEM((width,), jnp.int32),
            pltpu.VMEM((width,), jnp.int32),
        ] + [pltpu.SemaphoreType.DMA] * 7,
    )
    return kern(tracker, head_idx, compute_idx)
